# Initial kernel scaffold; baseline (speedup 1.0000x reference)
import jax
import jax.numpy as jnp
from jax.experimental import pallas as pl


def kernel(features, rois, spatial_scale):
    raise NotImplementedError("write your pallas kernel here")


# speedup vs baseline: 47.8123x; 47.8123x over previous
import functools

import jax
import jax.numpy as jnp
from jax import lax
from jax.experimental import pallas as pl
from jax.experimental.pallas import tpu as pltpu
from jax.experimental.pallas import tpu_sc as plsc

PH, PW = 7, 7
H = W = 50
C = 256
R = 1000
NB = PH * PW
NIDX = 4 * NB
IPAD = 256
NROWS = 200
SPLIT = 104
BPAD = 56
SC_R = BPAD * C


def _st2_body(f_ref, out_ref):
    f = f_ref[:]
    cb = f.shape[-1]
    neg = jnp.float32(-3.0e38)

    def shift_h(a, d):
        return jnp.concatenate(
            [a[d:], jnp.full((d, W, cb), neg, jnp.float32)], axis=0)

    def shift_w(a, d):
        return jnp.concatenate(
            [a[:, d:], jnp.full((H, d, cb), neg, jnp.float32)], axis=1)

    a = f
    for kh in range(4):
        if kh:
            a = jnp.maximum(a, shift_h(a, 1 << (kh - 1)))
        b = a
        for kw in range(4):
            if kw:
                b = jnp.maximum(b, shift_w(b, 1 << (kw - 1)))
            out_ref[kh, kw] = b


def _build_st2(feat_t):
    g = 2
    cb = C // g
    return pl.pallas_call(
        _st2_body,
        grid=(g,),
        in_specs=[pl.BlockSpec((H, W, cb), lambda i: (0, 0, i))],
        out_specs=pl.BlockSpec((4, 4, H, W, cb), lambda i: (0, 0, 0, 0, i)),
        out_shape=jax.ShapeDtypeStruct((4, 4, H, W, C), jnp.float32),
    )(feat_t)


def _idx_body(rois_ref, out_ref):
    rs = rois_ref[:]
    coords = jnp.round(rs).astype(jnp.int32)
    sw = coords[:, 0:1]
    sh = coords[:, 1:2]
    ew = coords[:, 2:3]
    eh = coords[:, 3:4]
    rw = jnp.maximum(ew - sw + 1, 1)
    rh = jnp.maximum(eh - sh + 1, 1)

    def edges(rv, start):
        m = jnp.arange(PH, dtype=jnp.int32)[None, :]
        lo = (m * rv) // PH
        hi = ((m + 1) * rv + (PH - 1)) // PH
        kk = jnp.arange(1, 31, dtype=jnp.int32)[None, :]
        p = jnp.sum((rv >= jnp.left_shift(jnp.int32(1), kk)).astype(jnp.int32),
                    axis=1, keepdims=True)
        t = (3 - p % 3) % 3
        tau = (jnp.left_shift(jnp.int32(1), t) * rv) % 7
        crit = ((rv % 7 != 0)
                & (4 * rv >= 7 * jnp.left_shift(jnp.int32(1), p))
                & (tau == 4)).astype(jnp.int32)
        adj = jnp.concatenate(
            [jnp.zeros((R, PH - 1), jnp.int32), crit], axis=1)
        hi = hi + adj
        lo = jnp.clip(lo + start, 0, H)
        hi = jnp.clip(hi + start, 0, H)
        return lo, hi

    lo_h, hi_h = edges(rh, sh)
    lo_w, hi_w = edges(rw, sw)

    def level(widths):
        return ((widths >= 2).astype(jnp.int32)
                + (widths >= 4).astype(jnp.int32)
                + (widths >= 8).astype(jnp.int32))

    kh = level(hi_h - lo_h)
    kw = level(hi_w - lo_w)
    h2 = hi_h - jnp.left_shift(jnp.int32(1), kh)
    w2 = hi_w - jnp.left_shift(jnp.int32(1), kw)

    base = (kh[:, :, None] * 4 + kw[:, None, :]) * (H * W)
    ah = lo_h[:, :, None] * W
    bh = h2[:, :, None] * W
    aw = lo_w[:, None, :]
    bw = w2[:, None, :]
    i0 = (base + ah + aw).reshape(R, NB)
    i1 = (base + ah + bw).reshape(R, NB)
    i2 = (base + bh + aw).reshape(R, NB)
    i3 = (base + bh + bw).reshape(R, NB)
    out_ref[:] = jnp.concatenate(
        [i0, i1, i2, i3, jnp.zeros((R, IPAD - NIDX), jnp.int32)], axis=1)


def _build_idx(rois_s):
    return pl.pallas_call(
        _idx_body,
        out_shape=jax.ShapeDtypeStruct((R, IPAD), jnp.int32),
    )(rois_s)


def _sc_pool(table_ref, idx_ref, out_ref,
             idxva, idxvb, gbufa, gbufb, outb, sa1, sa2, sb1, sb2):
    info = plsc.get_sparse_core_info()
    nw = info.num_cores * info.num_subcores
    wid = lax.axis_index("s") * info.num_cores + lax.axis_index("c")
    nt = (R + nw - 1) // nw

    def start_gather(idxv, gbuf, s1, s2):
        c1 = pltpu.async_copy(
            table_ref.at[idxv.at[pl.ds(0, SPLIT)]],
            gbuf.at[pl.ds(0, SPLIT)], s1)
        c2 = pltpu.async_copy(
            table_ref.at[idxv.at[pl.ds(SPLIT, NROWS - SPLIT)]],
            gbuf.at[pl.ds(SPLIT, NROWS - SPLIT)], s2)
        return c1, c2

    def compute(gbuf):
        def sub_body(s, carry2):
            cs = s * 16
            for b in range(NB):
                v = jnp.maximum(
                    jnp.maximum(gbuf[b, pl.ds(cs, 16)],
                                gbuf[NB + b, pl.ds(cs, 16)]),
                    jnp.maximum(gbuf[2 * NB + b, pl.ds(cs, 16)],
                                gbuf[3 * NB + b, pl.ds(cs, 16)]))
                outb[pl.ds(b * C + cs, 16)] = v
            return carry2

        lax.fori_loop(0, C // 16, sub_body, 0)

    r0p = wid
    pltpu.sync_copy(idx_ref.at[r0p], idxva)
    pa1, pa2 = start_gather(idxva, gbufa, sa1, sa2)

    def pair_body(t2, carry):
        r0 = (2 * t2) * nw + wid
        r1 = (2 * t2 + 1) * nw + wid
        r2 = (2 * t2 + 2) * nw + wid

        @pl.when(r1 < R)
        def _():
            pltpu.sync_copy(idx_ref.at[r1], idxvb)
            start_gather(idxvb, gbufb, sb1, sb2)

        @pl.when(r0 < R)
        def _():
            pltpu.make_async_copy(
                table_ref.at[idxva.at[pl.ds(0, SPLIT)]],
                gbufa.at[pl.ds(0, SPLIT)], sa1).wait()
            pltpu.make_async_copy(
                table_ref.at[idxva.at[pl.ds(SPLIT, NROWS - SPLIT)]],
                gbufa.at[pl.ds(SPLIT, NROWS - SPLIT)], sa2).wait()
            compute(gbufa)
            pltpu.sync_copy(outb, out_ref.at[pl.ds(r0 * SC_R, SC_R)])

        @pl.when(r2 < R)
        def _():
            pltpu.sync_copy(idx_ref.at[r2], idxva)
            start_gather(idxva, gbufa, sa1, sa2)

        @pl.when(r1 < R)
        def _():
            pltpu.make_async_copy(
                table_ref.at[idxvb.at[pl.ds(0, SPLIT)]],
                gbufb.at[pl.ds(0, SPLIT)], sb1).wait()
            pltpu.make_async_copy(
                table_ref.at[idxvb.at[pl.ds(SPLIT, NROWS - SPLIT)]],
                gbufb.at[pl.ds(SPLIT, NROWS - SPLIT)], sb2).wait()
            compute(gbufb)
            pltpu.sync_copy(outb, out_ref.at[pl.ds(r1 * SC_R, SC_R)])

        return carry

    lax.fori_loop(0, nt // 2, pair_body, 0)


def _sc_call(table, idx):
    mesh = plsc.VectorSubcoreMesh(core_axis_name="c", subcore_axis_name="s")
    kfn = functools.partial(
        pl.kernel,
        mesh=mesh,
        out_type=jax.ShapeDtypeStruct((R * SC_R,), jnp.float32),
        scratch_types=[
            pltpu.VMEM((IPAD,), jnp.int32),
            pltpu.VMEM((IPAD,), jnp.int32),
            pltpu.VMEM((NROWS, C), jnp.float32),
            pltpu.VMEM((NROWS, C), jnp.float32),
            pltpu.VMEM((SC_R,), jnp.float32),
            pltpu.SemaphoreType.DMA,
            pltpu.SemaphoreType.DMA,
            pltpu.SemaphoreType.DMA,
            pltpu.SemaphoreType.DMA,
        ],
    )(_sc_pool)
    return kfn(table, idx)


def _tr_body(x_ref, o_ref):
    o_ref[:] = jnp.swapaxes(x_ref[:], 1, 2)[:, :, :NB]


def _transpose_out(out_t):
    rb = 50
    return pl.pallas_call(
        _tr_body,
        grid=(R // rb,),
        in_specs=[pl.BlockSpec((rb, BPAD, C), lambda i: (i, 0, 0))],
        out_specs=pl.BlockSpec((rb, C, NB), lambda i: (i, 0, 0)),
        out_shape=jax.ShapeDtypeStruct((R, C, NB), jnp.float32),
    )(out_t)


def kernel(features, rois, spatial_scale):
    feats = jnp.asarray(features, jnp.float32).reshape(C, H * W)
    feat_t = feats.T.reshape(H, W, C)
    rois_s = (jnp.asarray(rois, jnp.float32)[:, 1:5]
              * jnp.float32(spatial_scale))
    st2 = _build_st2(feat_t)
    idx = _build_idx(rois_s)
    table = st2.reshape(4 * 4 * H * W, C)
    out_t = _sc_call(table, idx).reshape(R, BPAD, C)
    return _transpose_out(out_t).reshape(R, C, PH, PW)


# speedup vs baseline: 48.9443x; 1.0237x over previous
import functools

import jax
import jax.numpy as jnp
from jax import lax
from jax.experimental import pallas as pl
from jax.experimental.pallas import tpu as pltpu
from jax.experimental.pallas import tpu_sc as plsc

PH, PW = 7, 7
H = W = 50
C = 256
R = 1000
NB = PH * PW
NIDX = 4 * NB
IPAD = 256
NROWS = 200
SPLIT = 104
SC_R = NB * C


def _st2_body(f_ref, out_ref):
    f = f_ref[:]
    cb = f.shape[-1]
    neg = jnp.float32(-3.0e38)

    def shift_h(a, d):
        return jnp.concatenate(
            [a[d:], jnp.full((d, W, cb), neg, jnp.float32)], axis=0)

    def shift_w(a, d):
        return jnp.concatenate(
            [a[:, d:], jnp.full((H, d, cb), neg, jnp.float32)], axis=1)

    a = f
    for kh in range(4):
        if kh:
            a = jnp.maximum(a, shift_h(a, 1 << (kh - 1)))
        b = a
        for kw in range(4):
            if kw:
                b = jnp.maximum(b, shift_w(b, 1 << (kw - 1)))
            out_ref[kh, kw] = b


def _build_st2(feat_t):
    g = 2
    cb = C // g
    return pl.pallas_call(
        _st2_body,
        grid=(g,),
        in_specs=[pl.BlockSpec((H, W, cb), lambda i: (0, 0, i))],
        out_specs=pl.BlockSpec((4, 4, H, W, cb), lambda i: (0, 0, 0, 0, i)),
        out_shape=jax.ShapeDtypeStruct((4, 4, H, W, C), jnp.float32),
    )(feat_t)


def _idx_body(rois_ref, out_ref):
    rs = rois_ref[:]
    coords = jnp.round(rs).astype(jnp.int32)
    sw = coords[:, 0:1]
    sh = coords[:, 1:2]
    ew = coords[:, 2:3]
    eh = coords[:, 3:4]
    rw = jnp.maximum(ew - sw + 1, 1)
    rh = jnp.maximum(eh - sh + 1, 1)

    def edges(rv, start):
        m = jnp.arange(PH, dtype=jnp.int32)[None, :]
        lo = (m * rv) // PH
        hi = ((m + 1) * rv + (PH - 1)) // PH
        kk = jnp.arange(1, 31, dtype=jnp.int32)[None, :]
        p = jnp.sum((rv >= jnp.left_shift(jnp.int32(1), kk)).astype(jnp.int32),
                    axis=1, keepdims=True)
        t = (3 - p % 3) % 3
        tau = (jnp.left_shift(jnp.int32(1), t) * rv) % 7
        crit = ((rv % 7 != 0)
                & (4 * rv >= 7 * jnp.left_shift(jnp.int32(1), p))
                & (tau == 4)).astype(jnp.int32)
        adj = jnp.concatenate(
            [jnp.zeros((R, PH - 1), jnp.int32), crit], axis=1)
        hi = hi + adj
        lo = jnp.clip(lo + start, 0, H)
        hi = jnp.clip(hi + start, 0, H)
        return lo, hi

    lo_h, hi_h = edges(rh, sh)
    lo_w, hi_w = edges(rw, sw)

    def level(widths):
        return ((widths >= 2).astype(jnp.int32)
                + (widths >= 4).astype(jnp.int32)
                + (widths >= 8).astype(jnp.int32))

    kh = level(hi_h - lo_h)
    kw = level(hi_w - lo_w)
    h2 = hi_h - jnp.left_shift(jnp.int32(1), kh)
    w2 = hi_w - jnp.left_shift(jnp.int32(1), kw)

    base = (kh[:, :, None] * 4 + kw[:, None, :]) * (H * W)
    ah = lo_h[:, :, None] * W
    bh = h2[:, :, None] * W
    aw = lo_w[:, None, :]
    bw = w2[:, None, :]
    i0 = (base + ah + aw).reshape(R, NB)
    i1 = (base + ah + bw).reshape(R, NB)
    i2 = (base + bh + aw).reshape(R, NB)
    i3 = (base + bh + bw).reshape(R, NB)
    out_ref[:] = jnp.concatenate(
        [i0, i1, i2, i3, jnp.zeros((R, IPAD - NIDX), jnp.int32)], axis=1)


def _build_idx(rois_s):
    return pl.pallas_call(
        _idx_body,
        out_shape=jax.ShapeDtypeStruct((R, IPAD), jnp.int32),
    )(rois_s)


def _sc_pool(table_ref, idx_ref, out_ref,
             idxva, idxvb, gbufa, gbufb, outba, outbb,
             sa1, sa2, sb1, sb2, soa, sob):
    info = plsc.get_sparse_core_info()
    nw = info.num_cores * info.num_subcores
    wid = lax.axis_index("s") * info.num_cores + lax.axis_index("c")
    nt = (R + nw - 1) // nw

    def start_gather(idxv, gbuf, s1, s2):
        pltpu.async_copy(
            table_ref.at[idxv.at[pl.ds(0, SPLIT)]],
            gbuf.at[pl.ds(0, SPLIT)], s1)
        pltpu.async_copy(
            table_ref.at[idxv.at[pl.ds(SPLIT, NROWS - SPLIT)]],
            gbuf.at[pl.ds(SPLIT, NROWS - SPLIT)], s2)

    def wait_gather(idxv, gbuf, s1, s2):
        pltpu.make_async_copy(
            table_ref.at[idxv.at[pl.ds(0, SPLIT)]],
            gbuf.at[pl.ds(0, SPLIT)], s1).wait()
        pltpu.make_async_copy(
            table_ref.at[idxv.at[pl.ds(SPLIT, NROWS - SPLIT)]],
            gbuf.at[pl.ds(SPLIT, NROWS - SPLIT)], s2).wait()

    def compute(gbuf, outb):
        def sub_body(s, carry2):
            cs = s * 16
            for b in range(NB):
                v = jnp.maximum(
                    jnp.maximum(gbuf[b, pl.ds(cs, 16)],
                                gbuf[NB + b, pl.ds(cs, 16)]),
                    jnp.maximum(gbuf[2 * NB + b, pl.ds(cs, 16)],
                                gbuf[3 * NB + b, pl.ds(cs, 16)]))
                outb[pl.ds(b * C + cs, 16)] = v
            return carry2

        lax.fori_loop(0, C // 16, sub_body, 0)

    def drain_out(outb, so):
        pltpu.make_async_copy(
            outb, out_ref.at[pl.ds(wid * SC_R, SC_R)], so).wait()

    pltpu.sync_copy(idx_ref.at[wid], idxva)
    start_gather(idxva, gbufa, sa1, sa2)

    def pair_body(t2, carry):
        r0 = (2 * t2) * nw + wid
        r1 = (2 * t2 + 1) * nw + wid
        r2 = (2 * t2 + 2) * nw + wid

        @pl.when(r1 < R)
        def _():
            pltpu.sync_copy(idx_ref.at[r1], idxvb)
            start_gather(idxvb, gbufb, sb1, sb2)

        wait_gather(idxva, gbufa, sa1, sa2)

        @pl.when(t2 > 0)
        def _():
            drain_out(outba, soa)

        compute(gbufa, outba)
        pltpu.async_copy(outba, out_ref.at[pl.ds(r0 * SC_R, SC_R)], soa)

        @pl.when(r2 < R)
        def _():
            pltpu.sync_copy(idx_ref.at[r2], idxva)
            start_gather(idxva, gbufa, sa1, sa2)

        @pl.when(r1 < R)
        def _():
            wait_gather(idxvb, gbufb, sb1, sb2)

            @pl.when(t2 > 0)
            def _():
                drain_out(outbb, sob)

            compute(gbufb, outbb)
            pltpu.async_copy(outbb, out_ref.at[pl.ds(r1 * SC_R, SC_R)], sob)

        return carry

    lax.fori_loop(0, nt // 2, pair_body, 0)
    drain_out(outba, soa)
    drain_out(outbb, sob)


def _sc_call(table, idx):
    mesh = plsc.VectorSubcoreMesh(core_axis_name="c", subcore_axis_name="s")
    kfn = functools.partial(
        pl.kernel,
        mesh=mesh,
        out_type=jax.ShapeDtypeStruct((R * SC_R,), jnp.float32),
        scratch_types=[
            pltpu.VMEM((IPAD,), jnp.int32),
            pltpu.VMEM((IPAD,), jnp.int32),
            pltpu.VMEM((NROWS, C), jnp.float32),
            pltpu.VMEM((NROWS, C), jnp.float32),
            pltpu.VMEM((SC_R,), jnp.float32),
            pltpu.VMEM((SC_R,), jnp.float32),
            pltpu.SemaphoreType.DMA,
            pltpu.SemaphoreType.DMA,
            pltpu.SemaphoreType.DMA,
            pltpu.SemaphoreType.DMA,
            pltpu.SemaphoreType.DMA,
            pltpu.SemaphoreType.DMA,
        ],
    )(_sc_pool)
    return kfn(table, idx)


def _tr_body(x_ref, o_ref):
    o_ref[:] = jnp.swapaxes(x_ref[:], 1, 2)[:, :, :NB]


def _transpose_out(out_t):
    rb = 50
    return pl.pallas_call(
        _tr_body,
        grid=(R // rb,),
        in_specs=[pl.BlockSpec((rb, NB, C), lambda i: (i, 0, 0))],
        out_specs=pl.BlockSpec((rb, C, NB), lambda i: (i, 0, 0)),
        out_shape=jax.ShapeDtypeStruct((R, C, NB), jnp.float32),
    )(out_t)


def kernel(features, rois, spatial_scale):
    feats = jnp.asarray(features, jnp.float32).reshape(C, H * W)
    feat_t = feats.T.reshape(H, W, C)
    rois_s = (jnp.asarray(rois, jnp.float32)[:, 1:5]
              * jnp.float32(spatial_scale))
    st2 = _build_st2(feat_t)
    idx = _build_idx(rois_s)
    table = st2.reshape(4 * 4 * H * W, C)
    out_t = _sc_call(table, idx).reshape(R, NB, C)
    return _transpose_out(out_t).reshape(R, C, PH, PW)
